# Initial kernel scaffold; baseline (speedup 1.0000x reference)
import jax
import jax.numpy as jnp
from jax.experimental import pallas as pl


def kernel(x, edge_index, W_in, b_in, W0, b0, ln_g, ln_b, Wskip0, bskip0, W1, b1, Wskip1, bskip1):
    raise NotImplementedError("write your pallas kernel here")


# speedup vs baseline: 12.6945x; 12.6945x over previous
import functools

import jax
import jax.numpy as jnp
from jax import lax
from jax.experimental import pallas as pl
from jax.experimental.pallas import tpu as pltpu
from jax.experimental.pallas import tpu_sc as plsc

N = 10000
E = 320000
D = 128

NC = 2
NS = 16
NW = NC * NS
E_PER_W = E // NW
CH = 80
NCHUNK = E_PER_W // CH
NPAD = 10240
N_PER_TILE = NPAD // NS
ZR = 160

_mesh = functools.partial(
    plsc.VectorSubcoreMesh,
    core_axis_name="c",
    subcore_axis_name="s",
    num_cores=NC,
    num_subcores=NS,
)


def _spmm_body(u_hbm, row_hbm, col_hbm, zrows_hbm, out_hbm, acc, idxc, idxr, gbuf, gsem):
    cid = lax.axis_index("c")
    sid = lax.axis_index("s")
    wid = sid * NC + cid
    rbase = sid * N_PER_TILE
    for j in range(N_PER_TILE // ZR):
        pltpu.sync_copy(zrows_hbm, acc.at[pl.ds(rbase + j * ZR, ZR)])
    plsc.subcore_barrier()

    ebase = wid * E_PER_W

    def chunk(c, carry):
        off = ebase + c * CH
        pltpu.sync_copy(col_hbm.at[pl.ds(off, CH)], idxc)
        g = pltpu.async_copy(u_hbm.at[idxc], gbuf, gsem)
        pltpu.sync_copy(row_hbm.at[pl.ds(off, CH)], idxr)
        g.wait()
        pltpu.sync_copy(gbuf, acc.at[idxr], add=True)
        return carry

    lax.fori_loop(0, NCHUNK, chunk, 0)
    plsc.subcore_barrier()
    pltpu.sync_copy(acc.at[pl.ds(rbase, N_PER_TILE)],
                    out_hbm.at[cid, pl.ds(rbase, N_PER_TILE)])


def _spmm(u, row, col, zrows):
    return pl.kernel(
        _spmm_body,
        out_type=jax.ShapeDtypeStruct((NC, NPAD, D), jnp.float32),
        mesh=_mesh(),
        scratch_types=[
            pltpu.VMEM_SHARED((NPAD, D), jnp.float32),
            pltpu.VMEM((CH,), jnp.int32),
            pltpu.VMEM((CH,), jnp.int32),
            pltpu.VMEM((CH, D), jnp.float32),
            pltpu.SemaphoreType.DMA,
        ],
    )(u, row, col, zrows)


def _deg_body(row_hbm, zrows_hbm, ones_hbm, out_hbm, acc, idxr, obuf):
    cid = lax.axis_index("c")
    sid = lax.axis_index("s")
    wid = sid * NC + cid
    rbase = sid * N_PER_TILE
    for j in range(N_PER_TILE // ZR):
        pltpu.sync_copy(zrows_hbm, acc.at[pl.ds(rbase + j * ZR, ZR)])
    pltpu.sync_copy(ones_hbm, obuf)
    plsc.subcore_barrier()

    ebase = wid * E_PER_W

    def chunk(c, carry):
        off = ebase + c * CH
        pltpu.sync_copy(row_hbm.at[pl.ds(off, CH)], idxr)
        pltpu.sync_copy(obuf, acc.at[idxr], add=True)
        return carry

    lax.fori_loop(0, NCHUNK, chunk, 0)
    plsc.subcore_barrier()
    pltpu.sync_copy(acc.at[pl.ds(rbase, N_PER_TILE)],
                    out_hbm.at[cid, pl.ds(rbase, N_PER_TILE)])


def _deg(row, zrows, ones128):
    return pl.kernel(
        _deg_body,
        out_type=jax.ShapeDtypeStruct((NC, NPAD, D), jnp.float32),
        mesh=_mesh(),
        scratch_types=[
            pltpu.VMEM_SHARED((NPAD, D), jnp.float32),
            pltpu.VMEM((CH,), jnp.int32),
            pltpu.VMEM((CH, D), jnp.float32),
        ],
    )(row, zrows, ones128)


BLK = 1000
GRID = N // BLK

_f32 = jnp.float32


def _dinv_of(dp_ref):
    deg = dp_ref[0, :, 0:1] + dp_ref[1, :, 0:1] + 1.0
    return lax.rsqrt(deg)


def _kin_body(x_ref, w_ref, b_ref, dp_ref, h0_ref, u0_ref):
    h0 = jnp.dot(x_ref[...], w_ref[...], preferred_element_type=_f32) + b_ref[...]
    h0_ref[...] = h0
    u0_ref[...] = h0 * _dinv_of(dp_ref)


def _kmid_body(s_ref, u0_ref, h0_ref, dp_ref, w0_ref, b0_ref, g_ref, bb_ref,
               wsk_ref, bsk_ref, h_ref, u1_ref):
    dinv = _dinv_of(dp_ref)
    t = (s_ref[0] + s_ref[1] + u0_ref[...]) * dinv
    h = jnp.dot(t, w0_ref[...], preferred_element_type=_f32) + b0_ref[...]
    mu = jnp.mean(h, axis=1, keepdims=True)
    var = jnp.mean((h - mu) ** 2, axis=1, keepdims=True)
    h = (h - mu) * lax.rsqrt(var + 1e-5) * g_ref[...] + bb_ref[...]
    h = jnp.maximum(h, 0.0)
    h = h + jnp.dot(h0_ref[...], wsk_ref[...], preferred_element_type=_f32) + bsk_ref[...]
    h_ref[...] = h
    u1_ref[...] = h * dinv


def _kout_body(s_ref, u1_ref, h_ref, dp_ref, w1_ref, b1_ref, wsk_ref, bsk_ref, o_ref):
    dinv = _dinv_of(dp_ref)
    t = (s_ref[0] + s_ref[1] + u1_ref[...]) * dinv
    o_ref[...] = (jnp.dot(t, w1_ref[...], preferred_element_type=_f32) + b1_ref[...]
                  + jnp.dot(h_ref[...], wsk_ref[...], preferred_element_type=_f32)
                  + bsk_ref[...])


def _row_spec():
    return pl.BlockSpec((BLK, D), lambda i: (i, 0))


def _part_spec():
    return pl.BlockSpec((NC, BLK, D), lambda i: (0, i, 0))


def _dp_spec():
    return pl.BlockSpec((NC, BLK, D), lambda i: (0, i, 0))


def _w_spec():
    return pl.BlockSpec((D, D), lambda i: (0, 0))


def _b_spec():
    return pl.BlockSpec((1, D), lambda i: (0, 0))


def _sds(shape):
    return jax.ShapeDtypeStruct(shape, _f32)


def _kin(x, W_in, b_in, dp):
    return pl.pallas_call(
        _kin_body,
        grid=(GRID,),
        in_specs=[_row_spec(), _w_spec(), _b_spec(), _dp_spec()],
        out_specs=[_row_spec(), _row_spec()],
        out_shape=[_sds((N, D)), _sds((N, D))],
    )(x, W_in, b_in, dp)


def _kmid(s0, u0, h0, dp, W0, b0, ln_g, ln_b, Wskip0, bskip0):
    return pl.pallas_call(
        _kmid_body,
        grid=(GRID,),
        in_specs=[_part_spec(), _row_spec(), _row_spec(), _dp_spec(),
                  _w_spec(), _b_spec(), _b_spec(), _b_spec(), _w_spec(), _b_spec()],
        out_specs=[_row_spec(), _row_spec()],
        out_shape=[_sds((N, D)), _sds((N, D))],
    )(s0, u0, h0, dp, W0, b0, ln_g, ln_b, Wskip0, bskip0)


def _kout(s1, u1, h, dp, W1, b1, Wskip1, bskip1):
    return pl.pallas_call(
        _kout_body,
        grid=(GRID,),
        in_specs=[_part_spec(), _row_spec(), _row_spec(), _dp_spec(),
                  _w_spec(), _b_spec(), _w_spec(), _b_spec()],
        out_specs=_row_spec(),
        out_shape=_sds((N, D)),
    )(s1, u1, h, dp, W1, b1, Wskip1, bskip1)


def kernel(x, edge_index, W_in, b_in, W0, b0, ln_g, ln_b, Wskip0, bskip0,
           W1, b1, Wskip1, bskip1):
    ei = edge_index.astype(jnp.int32)
    row = ei[0]
    col = ei[1]
    zrows = jnp.zeros((ZR, D), _f32)
    ones128 = jnp.ones((CH, D), _f32)
    b_in2 = b_in.reshape(1, D)
    b02 = b0.reshape(1, D)
    b12 = b1.reshape(1, D)
    g2 = ln_g.reshape(1, D)
    lb2 = ln_b.reshape(1, D)
    bsk02 = bskip0.reshape(1, D)
    bsk12 = bskip1.reshape(1, D)

    dp = _deg(row, zrows, ones128)
    h0, u0 = _kin(x, W_in, b_in2, dp)
    s0 = _spmm(u0, row, col, zrows)
    h, u1 = _kmid(s0, u0, h0, dp, W0, b02, g2, lb2, Wskip0, bsk02)
    s1 = _spmm(u1, row, col, zrows)
    out = _kout(s1, u1, h, dp, W1, b12, Wskip1, bsk12)
    return out


# speedup vs baseline: 17.3535x; 1.3670x over previous
import functools

import jax
import jax.numpy as jnp
from jax import lax
from jax.experimental import pallas as pl
from jax.experimental.pallas import tpu as pltpu
from jax.experimental.pallas import tpu_sc as plsc

N = 10000
E = 320000
D = 128

NC = 2
NS = 16
NW = NC * NS
E_PER_W = E // NW
CH = 80
NCHUNK = E_PER_W // CH
NPAD = 10240
N_PER_TILE = NPAD // NS
ZR = 160

_mesh = functools.partial(
    plsc.VectorSubcoreMesh,
    core_axis_name="c",
    subcore_axis_name="s",
    num_cores=NC,
    num_subcores=NS,
)


NPAIR = (NCHUNK - 1) // 2


def _spmm_body(u_hbm, row_hbm, col_hbm, zrows_hbm, out_hbm, acc,
               idxc0, idxc1, idxr0, idxr1, gb0, gb1, sem0, sem1):
    cid = lax.axis_index("c")
    sid = lax.axis_index("s")
    wid = sid * NC + cid
    rbase = sid * N_PER_TILE
    for j in range(N_PER_TILE // ZR):
        pltpu.sync_copy(zrows_hbm, acc.at[pl.ds(rbase + j * ZR, ZR)])
    plsc.subcore_barrier()

    ebase = wid * E_PER_W

    pltpu.sync_copy(col_hbm.at[pl.ds(ebase, CH)], idxc0)
    pltpu.sync_copy(row_hbm.at[pl.ds(ebase, CH)], idxr0)
    g0 = pltpu.async_copy(u_hbm.at[idxc0], gb0, sem0)
    pltpu.sync_copy(col_hbm.at[pl.ds(ebase + CH, CH)], idxc1)
    pltpu.sync_copy(row_hbm.at[pl.ds(ebase + CH, CH)], idxr1)
    g1 = pltpu.async_copy(u_hbm.at[idxc1], gb1, sem1)

    def pair(i, carry):
        c2 = 2 * i + 2
        g0.wait()
        pltpu.sync_copy(gb0, acc.at[idxr0], add=True)
        off0 = ebase + c2 * CH
        pltpu.sync_copy(col_hbm.at[pl.ds(off0, CH)], idxc0)
        pltpu.sync_copy(row_hbm.at[pl.ds(off0, CH)], idxr0)
        pltpu.async_copy(u_hbm.at[idxc0], gb0, sem0)
        g1.wait()
        pltpu.sync_copy(gb1, acc.at[idxr1], add=True)

        @pl.when(i < NPAIR - 1)
        def _():
            off1 = ebase + (c2 + 1) * CH
            pltpu.sync_copy(col_hbm.at[pl.ds(off1, CH)], idxc1)
            pltpu.sync_copy(row_hbm.at[pl.ds(off1, CH)], idxr1)
            pltpu.async_copy(u_hbm.at[idxc1], gb1, sem1)

        return carry

    lax.fori_loop(0, NPAIR, pair, 0)
    g0.wait()
    pltpu.sync_copy(gb0, acc.at[idxr0], add=True)

    plsc.subcore_barrier()
    pltpu.sync_copy(acc.at[pl.ds(rbase, N_PER_TILE)],
                    out_hbm.at[cid, pl.ds(rbase, N_PER_TILE)])


def _spmm(u, row, col, zrows):
    return pl.kernel(
        _spmm_body,
        out_type=jax.ShapeDtypeStruct((NC, NPAD, D), jnp.float32),
        mesh=_mesh(),
        scratch_types=[
            pltpu.VMEM_SHARED((NPAD, D), jnp.float32),
            pltpu.VMEM((CH,), jnp.int32),
            pltpu.VMEM((CH,), jnp.int32),
            pltpu.VMEM((CH,), jnp.int32),
            pltpu.VMEM((CH,), jnp.int32),
            pltpu.VMEM((CH, D), jnp.float32),
            pltpu.VMEM((CH, D), jnp.float32),
            pltpu.SemaphoreType.DMA,
            pltpu.SemaphoreType.DMA,
        ],
    )(u, row, col, zrows)


def _deg_body(row_hbm, zrows_hbm, ones_hbm, out_hbm, acc, idxr0, idxr1, obuf,
              sem0, sem1):
    cid = lax.axis_index("c")
    sid = lax.axis_index("s")
    wid = sid * NC + cid
    rbase = sid * N_PER_TILE
    for j in range(N_PER_TILE // ZR):
        pltpu.sync_copy(zrows_hbm, acc.at[pl.ds(rbase + j * ZR, ZR)])
    pltpu.sync_copy(ones_hbm, obuf)
    plsc.subcore_barrier()

    ebase = wid * E_PER_W

    a0 = pltpu.async_copy(row_hbm.at[pl.ds(ebase, CH)], idxr0, sem0)
    a1 = pltpu.async_copy(row_hbm.at[pl.ds(ebase + CH, CH)], idxr1, sem1)

    def pair(i, carry):
        c2 = 2 * i + 2
        a0.wait()
        pltpu.sync_copy(obuf, acc.at[idxr0], add=True)
        pltpu.async_copy(row_hbm.at[pl.ds(ebase + c2 * CH, CH)], idxr0, sem0)
        a1.wait()
        pltpu.sync_copy(obuf, acc.at[idxr1], add=True)

        @pl.when(i < NPAIR - 1)
        def _():
            pltpu.async_copy(row_hbm.at[pl.ds(ebase + (c2 + 1) * CH, CH)],
                             idxr1, sem1)

        return carry

    lax.fori_loop(0, NPAIR, pair, 0)
    a0.wait()
    pltpu.sync_copy(obuf, acc.at[idxr0], add=True)

    plsc.subcore_barrier()
    pltpu.sync_copy(acc.at[pl.ds(rbase, N_PER_TILE)],
                    out_hbm.at[cid, pl.ds(rbase, N_PER_TILE)])


def _deg(row, zrows, ones128):
    return pl.kernel(
        _deg_body,
        out_type=jax.ShapeDtypeStruct((NC, NPAD, D), jnp.float32),
        mesh=_mesh(),
        scratch_types=[
            pltpu.VMEM_SHARED((NPAD, D), jnp.float32),
            pltpu.VMEM((CH,), jnp.int32),
            pltpu.VMEM((CH,), jnp.int32),
            pltpu.VMEM((CH, D), jnp.float32),
            pltpu.SemaphoreType.DMA,
            pltpu.SemaphoreType.DMA,
        ],
    )(row, zrows, ones128)


BLK = 1000
GRID = N // BLK

_f32 = jnp.float32


def _dinv_of(dp_ref):
    deg = dp_ref[0, :, 0:1] + dp_ref[1, :, 0:1] + 1.0
    return lax.rsqrt(deg)


def _kin_body(x_ref, w_ref, b_ref, dp_ref, h0_ref, u0_ref):
    h0 = jnp.dot(x_ref[...], w_ref[...], preferred_element_type=_f32) + b_ref[...]
    h0_ref[...] = h0
    u0_ref[...] = h0 * _dinv_of(dp_ref)


def _kmid_body(s_ref, u0_ref, h0_ref, dp_ref, w0_ref, b0_ref, g_ref, bb_ref,
               wsk_ref, bsk_ref, h_ref, u1_ref):
    dinv = _dinv_of(dp_ref)
    t = (s_ref[0] + s_ref[1] + u0_ref[...]) * dinv
    h = jnp.dot(t, w0_ref[...], preferred_element_type=_f32) + b0_ref[...]
    mu = jnp.mean(h, axis=1, keepdims=True)
    var = jnp.mean((h - mu) ** 2, axis=1, keepdims=True)
    h = (h - mu) * lax.rsqrt(var + 1e-5) * g_ref[...] + bb_ref[...]
    h = jnp.maximum(h, 0.0)
    h = h + jnp.dot(h0_ref[...], wsk_ref[...], preferred_element_type=_f32) + bsk_ref[...]
    h_ref[...] = h
    u1_ref[...] = h * dinv


def _kout_body(s_ref, u1_ref, h_ref, dp_ref, w1_ref, b1_ref, wsk_ref, bsk_ref, o_ref):
    dinv = _dinv_of(dp_ref)
    t = (s_ref[0] + s_ref[1] + u1_ref[...]) * dinv
    o_ref[...] = (jnp.dot(t, w1_ref[...], preferred_element_type=_f32) + b1_ref[...]
                  + jnp.dot(h_ref[...], wsk_ref[...], preferred_element_type=_f32)
                  + bsk_ref[...])


def _row_spec():
    return pl.BlockSpec((BLK, D), lambda i: (i, 0))


def _part_spec():
    return pl.BlockSpec((NC, BLK, D), lambda i: (0, i, 0))


def _dp_spec():
    return pl.BlockSpec((NC, BLK, D), lambda i: (0, i, 0))


def _w_spec():
    return pl.BlockSpec((D, D), lambda i: (0, 0))


def _b_spec():
    return pl.BlockSpec((1, D), lambda i: (0, 0))


def _sds(shape):
    return jax.ShapeDtypeStruct(shape, _f32)


def _kin(x, W_in, b_in, dp):
    return pl.pallas_call(
        _kin_body,
        grid=(GRID,),
        in_specs=[_row_spec(), _w_spec(), _b_spec(), _dp_spec()],
        out_specs=[_row_spec(), _row_spec()],
        out_shape=[_sds((N, D)), _sds((N, D))],
    )(x, W_in, b_in, dp)


def _kmid(s0, u0, h0, dp, W0, b0, ln_g, ln_b, Wskip0, bskip0):
    return pl.pallas_call(
        _kmid_body,
        grid=(GRID,),
        in_specs=[_part_spec(), _row_spec(), _row_spec(), _dp_spec(),
                  _w_spec(), _b_spec(), _b_spec(), _b_spec(), _w_spec(), _b_spec()],
        out_specs=[_row_spec(), _row_spec()],
        out_shape=[_sds((N, D)), _sds((N, D))],
    )(s0, u0, h0, dp, W0, b0, ln_g, ln_b, Wskip0, bskip0)


def _kout(s1, u1, h, dp, W1, b1, Wskip1, bskip1):
    return pl.pallas_call(
        _kout_body,
        grid=(GRID,),
        in_specs=[_part_spec(), _row_spec(), _row_spec(), _dp_spec(),
                  _w_spec(), _b_spec(), _w_spec(), _b_spec()],
        out_specs=_row_spec(),
        out_shape=_sds((N, D)),
    )(s1, u1, h, dp, W1, b1, Wskip1, bskip1)


def kernel(x, edge_index, W_in, b_in, W0, b0, ln_g, ln_b, Wskip0, bskip0,
           W1, b1, Wskip1, bskip1):
    ei = edge_index.astype(jnp.int32)
    row = ei[0]
    col = ei[1]
    zrows = jnp.zeros((ZR, D), _f32)
    ones128 = jnp.ones((CH, D), _f32)
    b_in2 = b_in.reshape(1, D)
    b02 = b0.reshape(1, D)
    b12 = b1.reshape(1, D)
    g2 = ln_g.reshape(1, D)
    lb2 = ln_b.reshape(1, D)
    bsk02 = bskip0.reshape(1, D)
    bsk12 = bskip1.reshape(1, D)

    dp = _deg(row, zrows, ones128)
    h0, u0 = _kin(x, W_in, b_in2, dp)
    s0 = _spmm(u0, row, col, zrows)
    h, u1 = _kmid(s0, u0, h0, dp, W0, b02, g2, lb2, Wskip0, bsk02)
    s1 = _spmm(u1, row, col, zrows)
    out = _kout(s1, u1, h, dp, W1, b12, Wskip1, bsk12)
    return out


# speedup vs baseline: 22.7192x; 1.3092x over previous
import functools

import jax
import jax.numpy as jnp
from jax import lax
from jax.experimental import pallas as pl
from jax.experimental.pallas import tpu as pltpu
from jax.experimental.pallas import tpu_sc as plsc

N = 10000
E = 320000
D = 128

NC = 2
NS = 16
NW = NC * NS
E_PER_W = E // NW
CH = 80
NCHUNK = E_PER_W // CH
NPAD = 10240
N_PER_TILE = NPAD // NS
ZR = 160

_mesh = functools.partial(
    plsc.VectorSubcoreMesh,
    core_axis_name="c",
    subcore_axis_name="s",
    num_cores=NC,
    num_subcores=NS,
)


NQUAD = (NCHUNK - 1) // 4


def _spmm_body(u_hbm, row_hbm, col_hbm, zrows_hbm, out_hbm, acc, cbuf,
               gb0, gb1, ir0, ir1, ir2, ir3,
               gsem0, gsem1, isem0, isem1, isem2, isem3):
    cid = lax.axis_index("c")
    sid = lax.axis_index("s")
    wid = sid * NC + cid
    rbase = sid * N_PER_TILE
    for j in range(N_PER_TILE // ZR):
        pltpu.sync_copy(zrows_hbm, acc.at[pl.ds(rbase + j * ZR, ZR)])

    pltpu.sync_copy(col_hbm.at[wid], cbuf)
    plsc.subcore_barrier()

    irs = [ir0, ir1, ir2, ir3]
    isems = [isem0, isem1, isem2, isem3]
    a = [pltpu.async_copy(row_hbm.at[wid, b], irs[b], isems[b])
         for b in range(4)]
    g0 = pltpu.async_copy(u_hbm.at[cbuf.at[0]], gb0, gsem0)
    g1 = pltpu.async_copy(u_hbm.at[cbuf.at[1]], gb1, gsem1)
    gs = [g0, g1]

    def quad(i, carry):
        c = 4 * i
        for p in range(4):
            gb = (gb0, gb1)[p % 2]
            gs[p % 2].wait()
            a[p].wait()
            pltpu.sync_copy(gb, acc.at[irs[p]], add=True)
            nxt_g = c + p + 2
            if p < 2:
                pltpu.async_copy(u_hbm.at[cbuf.at[nxt_g]], gb,
                                 (gsem0, gsem1)[p % 2])
            else:
                @pl.when(nxt_g < NCHUNK)
                def _():
                    pltpu.async_copy(u_hbm.at[cbuf.at[nxt_g]], gb,
                                     (gsem0, gsem1)[p % 2])
            nxt_i = c + p + 4
            if p == 0:
                pltpu.async_copy(row_hbm.at[wid, nxt_i], irs[p], isems[p])
            else:
                @pl.when(nxt_i < NCHUNK)
                def _():
                    pltpu.async_copy(row_hbm.at[wid, nxt_i], irs[p], isems[p])

        return carry

    lax.fori_loop(0, NQUAD, quad, 0)
    g0.wait()
    a[0].wait()
    pltpu.sync_copy(gb0, acc.at[ir0], add=True)

    plsc.subcore_barrier()
    pltpu.sync_copy(acc.at[pl.ds(rbase, N_PER_TILE)],
                    out_hbm.at[cid, pl.ds(rbase, N_PER_TILE)])


def _spmm(u, row3d, col3d, zrows):
    return pl.kernel(
        _spmm_body,
        out_type=jax.ShapeDtypeStruct((NC, NPAD, D), jnp.float32),
        mesh=_mesh(),
        scratch_types=[
            pltpu.VMEM_SHARED((NPAD, D), jnp.float32),
            pltpu.VMEM((NCHUNK, CH), jnp.int32),
            pltpu.VMEM((CH, D), jnp.float32),
            pltpu.VMEM((CH, D), jnp.float32),
            pltpu.VMEM((CH,), jnp.int32),
            pltpu.VMEM((CH,), jnp.int32),
            pltpu.VMEM((CH,), jnp.int32),
            pltpu.VMEM((CH,), jnp.int32),
            pltpu.SemaphoreType.DMA,
            pltpu.SemaphoreType.DMA,
            pltpu.SemaphoreType.DMA,
            pltpu.SemaphoreType.DMA,
            pltpu.SemaphoreType.DMA,
            pltpu.SemaphoreType.DMA,
        ],
    )(u, row3d, col3d, zrows)


def _deg_body(row_hbm, zrows_hbm, ones_hbm, out_hbm, acc, rbuf, obuf):
    cid = lax.axis_index("c")
    sid = lax.axis_index("s")
    wid = sid * NC + cid
    rbase = sid * N_PER_TILE
    for j in range(N_PER_TILE // ZR):
        pltpu.sync_copy(zrows_hbm, acc.at[pl.ds(rbase + j * ZR, ZR)])
    pltpu.sync_copy(ones_hbm, obuf)
    pltpu.sync_copy(row_hbm.at[wid], rbuf)
    plsc.subcore_barrier()

    def chunk(c, carry):
        pltpu.sync_copy(obuf, acc.at[rbuf.at[c]], add=True)
        return carry

    lax.fori_loop(0, NCHUNK, chunk, 0)

    plsc.subcore_barrier()
    pltpu.sync_copy(acc.at[pl.ds(rbase, N_PER_TILE)],
                    out_hbm.at[cid, pl.ds(rbase, N_PER_TILE)])


def _deg(row3d, zrows, ones128):
    return pl.kernel(
        _deg_body,
        out_type=jax.ShapeDtypeStruct((NC, NPAD, D), jnp.float32),
        mesh=_mesh(),
        scratch_types=[
            pltpu.VMEM_SHARED((NPAD, D), jnp.float32),
            pltpu.VMEM((NCHUNK, CH), jnp.int32),
            pltpu.VMEM((CH, D), jnp.float32),
        ],
    )(row3d, zrows, ones128)


BLK = 1000
GRID = N // BLK

_f32 = jnp.float32


def _dinv_of(dp_ref):
    deg = dp_ref[0, :, 0:1] + dp_ref[1, :, 0:1] + 1.0
    return lax.rsqrt(deg)


def _kin_body(x_ref, w_ref, b_ref, dp_ref, h0_ref, u0_ref):
    h0 = jnp.dot(x_ref[...], w_ref[...], preferred_element_type=_f32) + b_ref[...]
    h0_ref[...] = h0
    u0_ref[...] = h0 * _dinv_of(dp_ref)


def _kmid_body(s_ref, u0_ref, h0_ref, dp_ref, w0_ref, b0_ref, g_ref, bb_ref,
               wsk_ref, bsk_ref, h_ref, u1_ref):
    dinv = _dinv_of(dp_ref)
    t = (s_ref[0] + s_ref[1] + u0_ref[...]) * dinv
    h = jnp.dot(t, w0_ref[...], preferred_element_type=_f32) + b0_ref[...]
    mu = jnp.mean(h, axis=1, keepdims=True)
    var = jnp.mean((h - mu) ** 2, axis=1, keepdims=True)
    h = (h - mu) * lax.rsqrt(var + 1e-5) * g_ref[...] + bb_ref[...]
    h = jnp.maximum(h, 0.0)
    h = h + jnp.dot(h0_ref[...], wsk_ref[...], preferred_element_type=_f32) + bsk_ref[...]
    h_ref[...] = h
    u1_ref[...] = h * dinv


def _kout_body(s_ref, u1_ref, h_ref, dp_ref, w1_ref, b1_ref, wsk_ref, bsk_ref, o_ref):
    dinv = _dinv_of(dp_ref)
    t = (s_ref[0] + s_ref[1] + u1_ref[...]) * dinv
    o_ref[...] = (jnp.dot(t, w1_ref[...], preferred_element_type=_f32) + b1_ref[...]
                  + jnp.dot(h_ref[...], wsk_ref[...], preferred_element_type=_f32)
                  + bsk_ref[...])


def _row_spec():
    return pl.BlockSpec((BLK, D), lambda i: (i, 0))


def _part_spec():
    return pl.BlockSpec((NC, BLK, D), lambda i: (0, i, 0))


def _dp_spec():
    return pl.BlockSpec((NC, BLK, D), lambda i: (0, i, 0))


def _w_spec():
    return pl.BlockSpec((D, D), lambda i: (0, 0))


def _b_spec():
    return pl.BlockSpec((1, D), lambda i: (0, 0))


def _sds(shape):
    return jax.ShapeDtypeStruct(shape, _f32)


def _kin(x, W_in, b_in, dp):
    return pl.pallas_call(
        _kin_body,
        grid=(GRID,),
        in_specs=[_row_spec(), _w_spec(), _b_spec(), _dp_spec()],
        out_specs=[_row_spec(), _row_spec()],
        out_shape=[_sds((N, D)), _sds((N, D))],
    )(x, W_in, b_in, dp)


def _kmid(s0, u0, h0, dp, W0, b0, ln_g, ln_b, Wskip0, bskip0):
    return pl.pallas_call(
        _kmid_body,
        grid=(GRID,),
        in_specs=[_part_spec(), _row_spec(), _row_spec(), _dp_spec(),
                  _w_spec(), _b_spec(), _b_spec(), _b_spec(), _w_spec(), _b_spec()],
        out_specs=[_row_spec(), _row_spec()],
        out_shape=[_sds((N, D)), _sds((N, D))],
    )(s0, u0, h0, dp, W0, b0, ln_g, ln_b, Wskip0, bskip0)


def _kout(s1, u1, h, dp, W1, b1, Wskip1, bskip1):
    return pl.pallas_call(
        _kout_body,
        grid=(GRID,),
        in_specs=[_part_spec(), _row_spec(), _row_spec(), _dp_spec(),
                  _w_spec(), _b_spec(), _w_spec(), _b_spec()],
        out_specs=_row_spec(),
        out_shape=_sds((N, D)),
    )(s1, u1, h, dp, W1, b1, Wskip1, bskip1)


def kernel(x, edge_index, W_in, b_in, W0, b0, ln_g, ln_b, Wskip0, bskip0,
           W1, b1, Wskip1, bskip1):
    ei = edge_index.astype(jnp.int32)
    row3d = ei[0].reshape(NW, NCHUNK, CH)
    col3d = ei[1].reshape(NW, NCHUNK, CH)
    zrows = jnp.zeros((ZR, D), _f32)
    ones128 = jnp.ones((CH, D), _f32)
    b_in2 = b_in.reshape(1, D)
    b02 = b0.reshape(1, D)
    b12 = b1.reshape(1, D)
    g2 = ln_g.reshape(1, D)
    lb2 = ln_b.reshape(1, D)
    bsk02 = bskip0.reshape(1, D)
    bsk12 = bskip1.reshape(1, D)

    dp = _deg(row3d, zrows, ones128)
    h0, u0 = _kin(x, W_in, b_in2, dp)
    s0 = _spmm(u0, row3d, col3d, zrows)
    h, u1 = _kmid(s0, u0, h0, dp, W0, b02, g2, lb2, Wskip0, bsk02)
    s1 = _spmm(u1, row3d, col3d, zrows)
    out = _kout(s1, u1, h, dp, W1, b12, Wskip1, bsk12)
    return out


# speedup vs baseline: 25.4074x; 1.1183x over previous
import functools

import jax
import jax.numpy as jnp
from jax import lax
from jax.experimental import pallas as pl
from jax.experimental.pallas import tpu as pltpu
from jax.experimental.pallas import tpu_sc as plsc

N = 10000
E = 320000
D = 128

NC = 2
NS = 16
NW = NC * NS
E_PER_W = E // NW
CH = 80
NCHUNK = E_PER_W // CH
NPAD = 10240
N_PER_TILE = NPAD // NS
ZR = 160

_mesh = functools.partial(
    plsc.VectorSubcoreMesh,
    core_axis_name="c",
    subcore_axis_name="s",
    num_cores=NC,
    num_subcores=NS,
)


NTRI = (NCHUNK - 2) // 3


def _spmm_body(u_hbm, row_hbm, col_hbm, zrows_hbm, out_hbm, acc, cbuf,
               gb0, gb1, gb2, ir0, ir1, ir2,
               gsem0, gsem1, gsem2, isem0, isem1, isem2):
    cid = lax.axis_index("c")
    sid = lax.axis_index("s")
    wid = sid * NC + cid
    rbase = sid * N_PER_TILE
    for j in range(N_PER_TILE // ZR):
        pltpu.sync_copy(zrows_hbm, acc.at[pl.ds(rbase + j * ZR, ZR)])

    pltpu.sync_copy(col_hbm.at[wid], cbuf)
    plsc.subcore_barrier()

    gbs = [gb0, gb1, gb2]
    gsems = [gsem0, gsem1, gsem2]
    irs = [ir0, ir1, ir2]
    isems = [isem0, isem1, isem2]
    a = [pltpu.async_copy(row_hbm.at[wid, b], irs[b], isems[b])
         for b in range(3)]
    gs = [pltpu.async_copy(u_hbm.at[cbuf.at[b]], gbs[b], gsems[b])
          for b in range(3)]

    def tri(i, carry):
        c = 3 * i
        for p in range(3):
            gs[p].wait()
            a[p].wait()
            pltpu.sync_copy(gbs[p], acc.at[irs[p]], add=True)
            nxt = c + p + 3
            if p < 2:
                pltpu.async_copy(u_hbm.at[cbuf.at[nxt]], gbs[p], gsems[p])
                pltpu.async_copy(row_hbm.at[wid, nxt], irs[p], isems[p])
            else:
                @pl.when(nxt < NCHUNK)
                def _():
                    pltpu.async_copy(u_hbm.at[cbuf.at[nxt]], gbs[p], gsems[p])
                    pltpu.async_copy(row_hbm.at[wid, nxt], irs[p], isems[p])

        return carry

    lax.fori_loop(0, NTRI, tri, 0)
    for p in range(2):
        gs[p].wait()
        a[p].wait()
        pltpu.sync_copy(gbs[p], acc.at[irs[p]], add=True)

    plsc.subcore_barrier()
    pltpu.sync_copy(acc.at[pl.ds(rbase, N_PER_TILE)],
                    out_hbm.at[cid, pl.ds(rbase, N_PER_TILE)])


def _spmm(u, row3d, col3d, zrows):
    return pl.kernel(
        _spmm_body,
        out_type=jax.ShapeDtypeStruct((NC, NPAD, D), jnp.float32),
        mesh=_mesh(),
        scratch_types=[
            pltpu.VMEM_SHARED((NPAD, D), jnp.float32),
            pltpu.VMEM((NCHUNK, CH), jnp.int32),
            pltpu.VMEM((CH, D), jnp.float32),
            pltpu.VMEM((CH, D), jnp.float32),
            pltpu.VMEM((CH, D), jnp.float32),
            pltpu.VMEM((CH,), jnp.int32),
            pltpu.VMEM((CH,), jnp.int32),
            pltpu.VMEM((CH,), jnp.int32),
            pltpu.SemaphoreType.DMA,
            pltpu.SemaphoreType.DMA,
            pltpu.SemaphoreType.DMA,
            pltpu.SemaphoreType.DMA,
            pltpu.SemaphoreType.DMA,
            pltpu.SemaphoreType.DMA,
        ],
    )(u, row3d, col3d, zrows)


def _deg_body(row_hbm, zrows_hbm, ones_hbm, out_hbm, acc, rbuf, obuf):
    cid = lax.axis_index("c")
    sid = lax.axis_index("s")
    wid = sid * NC + cid
    rbase = sid * N_PER_TILE
    for j in range(N_PER_TILE // ZR):
        pltpu.sync_copy(zrows_hbm, acc.at[pl.ds(rbase + j * ZR, ZR)])
    pltpu.sync_copy(ones_hbm, obuf)
    pltpu.sync_copy(row_hbm.at[wid], rbuf)
    plsc.subcore_barrier()

    def chunk(c, carry):
        pltpu.sync_copy(obuf, acc.at[rbuf.at[c]], add=True)
        return carry

    lax.fori_loop(0, NCHUNK, chunk, 0)

    plsc.subcore_barrier()
    pltpu.sync_copy(acc.at[pl.ds(rbase, N_PER_TILE)],
                    out_hbm.at[cid, pl.ds(rbase, N_PER_TILE)])


def _deg(row3d, zrows, ones128):
    return pl.kernel(
        _deg_body,
        out_type=jax.ShapeDtypeStruct((NC, NPAD, D), jnp.float32),
        mesh=_mesh(),
        scratch_types=[
            pltpu.VMEM_SHARED((NPAD, D), jnp.float32),
            pltpu.VMEM((NCHUNK, CH), jnp.int32),
            pltpu.VMEM((CH, D), jnp.float32),
        ],
    )(row3d, zrows, ones128)


BLK = 1000
GRID = N // BLK

_f32 = jnp.float32


def _dinv_of(dp_ref):
    deg = dp_ref[0, :, 0:1] + dp_ref[1, :, 0:1] + 1.0
    return lax.rsqrt(deg)


def _kin_body(x_ref, w_ref, b_ref, dp_ref, h0_ref, u0_ref):
    h0 = jnp.dot(x_ref[...], w_ref[...], preferred_element_type=_f32) + b_ref[...]
    h0_ref[...] = h0
    u0_ref[...] = h0 * _dinv_of(dp_ref)


def _kmid_body(s_ref, u0_ref, h0_ref, dp_ref, w0_ref, b0_ref, g_ref, bb_ref,
               wsk_ref, bsk_ref, h_ref, u1_ref):
    dinv = _dinv_of(dp_ref)
    t = (s_ref[0] + s_ref[1] + u0_ref[...]) * dinv
    h = jnp.dot(t, w0_ref[...], preferred_element_type=_f32) + b0_ref[...]
    mu = jnp.mean(h, axis=1, keepdims=True)
    var = jnp.mean((h - mu) ** 2, axis=1, keepdims=True)
    h = (h - mu) * lax.rsqrt(var + 1e-5) * g_ref[...] + bb_ref[...]
    h = jnp.maximum(h, 0.0)
    h = h + jnp.dot(h0_ref[...], wsk_ref[...], preferred_element_type=_f32) + bsk_ref[...]
    h_ref[...] = h
    u1_ref[...] = h * dinv


def _kout_body(s_ref, u1_ref, h_ref, dp_ref, w1_ref, b1_ref, wsk_ref, bsk_ref, o_ref):
    dinv = _dinv_of(dp_ref)
    t = (s_ref[0] + s_ref[1] + u1_ref[...]) * dinv
    o_ref[...] = (jnp.dot(t, w1_ref[...], preferred_element_type=_f32) + b1_ref[...]
                  + jnp.dot(h_ref[...], wsk_ref[...], preferred_element_type=_f32)
                  + bsk_ref[...])


def _row_spec():
    return pl.BlockSpec((BLK, D), lambda i: (i, 0))


def _part_spec():
    return pl.BlockSpec((NC, BLK, D), lambda i: (0, i, 0))


def _dp_spec():
    return pl.BlockSpec((NC, BLK, D), lambda i: (0, i, 0))


def _w_spec():
    return pl.BlockSpec((D, D), lambda i: (0, 0))


def _b_spec():
    return pl.BlockSpec((1, D), lambda i: (0, 0))


def _sds(shape):
    return jax.ShapeDtypeStruct(shape, _f32)


def _kin(x, W_in, b_in, dp):
    return pl.pallas_call(
        _kin_body,
        grid=(GRID,),
        in_specs=[_row_spec(), _w_spec(), _b_spec(), _dp_spec()],
        out_specs=[_row_spec(), _row_spec()],
        out_shape=[_sds((N, D)), _sds((N, D))],
    )(x, W_in, b_in, dp)


def _kmid(s0, u0, h0, dp, W0, b0, ln_g, ln_b, Wskip0, bskip0):
    return pl.pallas_call(
        _kmid_body,
        grid=(GRID,),
        in_specs=[_part_spec(), _row_spec(), _row_spec(), _dp_spec(),
                  _w_spec(), _b_spec(), _b_spec(), _b_spec(), _w_spec(), _b_spec()],
        out_specs=[_row_spec(), _row_spec()],
        out_shape=[_sds((N, D)), _sds((N, D))],
    )(s0, u0, h0, dp, W0, b0, ln_g, ln_b, Wskip0, bskip0)


def _kout(s1, u1, h, dp, W1, b1, Wskip1, bskip1):
    return pl.pallas_call(
        _kout_body,
        grid=(GRID,),
        in_specs=[_part_spec(), _row_spec(), _row_spec(), _dp_spec(),
                  _w_spec(), _b_spec(), _w_spec(), _b_spec()],
        out_specs=_row_spec(),
        out_shape=_sds((N, D)),
    )(s1, u1, h, dp, W1, b1, Wskip1, bskip1)


def kernel(x, edge_index, W_in, b_in, W0, b0, ln_g, ln_b, Wskip0, bskip0,
           W1, b1, Wskip1, bskip1):
    ei = edge_index.astype(jnp.int32)
    row3d = ei[0].reshape(NW, NCHUNK, CH)
    col3d = ei[1].reshape(NW, NCHUNK, CH)
    zrows = jnp.zeros((ZR, D), _f32)
    ones128 = jnp.ones((CH, D), _f32)
    b_in2 = b_in.reshape(1, D)
    b02 = b0.reshape(1, D)
    b12 = b1.reshape(1, D)
    g2 = ln_g.reshape(1, D)
    lb2 = ln_b.reshape(1, D)
    bsk02 = bskip0.reshape(1, D)
    bsk12 = bskip1.reshape(1, D)

    dp = _deg(row3d, zrows, ones128)
    h0, u0 = _kin(x, W_in, b_in2, dp)
    s0 = _spmm(u0, row3d, col3d, zrows)
    h, u1 = _kmid(s0, u0, h0, dp, W0, b02, g2, lb2, Wskip0, bsk02)
    s1 = _spmm(u1, row3d, col3d, zrows)
    out = _kout(s1, u1, h, dp, W1, b12, Wskip1, bsk12)
    return out
